# baseline (speedup 1.0000x reference)
import jax
import jax.numpy as jnp
from jax.experimental import pallas as pl
from jax.experimental.pallas import tpu as pltpu

_NTOKEN = 50000
_NINP = 256
_NHID = 256
_NHEADS = 8
_NLAYERS = 6
_B = 100
_NPG = 100
_EPG = 1600
_HD = _NHID // _NHEADS


def _gnn_graph_kernel(src_ref, dst_ref, x_ref, w_ref, as_ref, ad_ref, b_ref,
                      out_ref, acc_ref):

    src = src_ref[0]
    dst = dst_ref[0]

    node_iota = jax.lax.broadcasted_iota(jnp.int32, (_NPG, _EPG), 0)
    st = (node_iota == src).astype(jnp.bfloat16)
    dt = (node_iota == dst).astype(jnp.bfloat16)
    cnt = jax.lax.dot_general(dt, st, (((1,), (1,)), ((), ())),
                              preferred_element_type=jnp.float32)
    eye_r = jax.lax.broadcasted_iota(jnp.int32, (_NPG, _NPG), 0)
    eye_c = jax.lax.broadcasted_iota(jnp.int32, (_NPG, _NPG), 1)
    cnt = cnt + (eye_r == eye_c).astype(jnp.float32)
    mask = cnt > 0.5

    ones_col = jnp.ones((_NPG, 1), dtype=jnp.bfloat16)
    x = x_ref[0]
    for l in range(_NLAYERS):
        if l > 0:
            x = jnp.maximum(x, 0.0)
        w = w_ref[l]
        a_s = as_ref[l]
        a_d = ad_ref[l]
        h = jnp.dot(x.astype(jnp.bfloat16), w,
                    preferred_element_type=jnp.float32)
        hb = h.astype(jnp.bfloat16)
        als_t = jax.lax.dot_general(a_s, hb, (((0,), (1,)), ((), ())),
                                    preferred_element_type=jnp.float32)
        ald = jnp.dot(hb, a_d, preferred_element_type=jnp.float32)
        ea_st = jnp.exp(als_t)
        eb_st = jnp.exp(0.2 * als_t)
        ea_d = jnp.exp(ald)
        eb_d = jnp.exp(0.2 * ald)
        for hd_i in range(_NHEADS):
            cond = (ald[:, hd_i:hd_i + 1] + als_t[hd_i:hd_i + 1, :]) > 0
            pa = ea_d[:, hd_i:hd_i + 1] * ea_st[hd_i:hd_i + 1, :]
            pb = eb_d[:, hd_i:hd_i + 1] * eb_st[hd_i:hd_i + 1, :]
            ee = cnt * jnp.where(cond, pa, pb)
            denom = jnp.sum(ee, axis=1, keepdims=True)
            h_head = hb[:, hd_i * _HD:(hd_i + 1) * _HD]
            agg = jnp.dot(ee.astype(jnp.bfloat16), h_head,
                          preferred_element_type=jnp.float32)
            acc_ref[:, hd_i * _HD:(hd_i + 1) * _HD] = agg / denom
        x = acc_ref[...] + b_ref[l]

    out_ref[0] = jnp.sum(x, axis=0, keepdims=True) * (1.0 / _NPG)


def kernel(nodes, edges, emb,
           W0, as0, ad0, b0, W1, as1, ad1, b1, W2, as2, ad2, b2,
           W3, as3, ad3, b3, W4, as4, ad4, b4, W5, as5, ad5, b5):
    x = emb[nodes.reshape(-1)].reshape(_B, _NPG, _NINP)
    src = edges[:, 0::2].reshape(_B, 1, _EPG)
    dst = edges[:, 1::2].reshape(_B, 1, _EPG)

    w_all = jnp.stack([W0, W1, W2, W3, W4, W5]).astype(jnp.bfloat16)
    eye_h = jnp.eye(_NHEADS, dtype=jnp.float32)
    as_all = jnp.stack([
        jnp.einsum('hj,hk->hjk', a, eye_h).reshape(_NHID, _NHEADS)
        for a in (as0, as1, as2, as3, as4, as5)]).astype(jnp.bfloat16)
    ad_all = jnp.stack([
        jnp.einsum('hj,hk->hjk', a, eye_h).reshape(_NHID, _NHEADS)
        for a in (ad0, ad1, ad2, ad3, ad4, ad5)]).astype(jnp.bfloat16)
    b_all = jnp.stack([b0, b1, b2, b3, b4, b5]).reshape(_NLAYERS, 1, _NHID)

    out = pl.pallas_call(
        _gnn_graph_kernel,
        grid=(_B,),
        in_specs=[
            pl.BlockSpec((1, 1, _EPG), lambda g: (g, 0, 0)),
            pl.BlockSpec((1, 1, _EPG), lambda g: (g, 0, 0)),
            pl.BlockSpec((1, _NPG, _NINP), lambda g: (g, 0, 0)),
            pl.BlockSpec((_NLAYERS, _NINP, _NHID), lambda g: (0, 0, 0)),
            pl.BlockSpec((_NLAYERS, _NHID, _NHEADS), lambda g: (0, 0, 0)),
            pl.BlockSpec((_NLAYERS, _NHID, _NHEADS), lambda g: (0, 0, 0)),
            pl.BlockSpec((_NLAYERS, 1, _NHID), lambda g: (0, 0, 0)),
        ],
        out_specs=pl.BlockSpec((1, 1, _NHID), lambda g: (g, 0, 0)),
        out_shape=jax.ShapeDtypeStruct((_B, 1, _NHID), jnp.float32),
        scratch_shapes=[pltpu.VMEM((_NPG, _NHID), jnp.float32)],
        compiler_params=pltpu.CompilerParams(
            dimension_semantics=('parallel',)),
    )(src, dst, x, w_all, as_all, ad_all, b_all)
    return out.reshape(_B, _NHID)


# speedup vs baseline: 2.4801x; 2.4801x over previous
import jax
import jax.numpy as jnp
from jax.experimental import pallas as pl
from jax.experimental.pallas import tpu as pltpu

_NTOKEN = 50000
_NINP = 256
_NHID = 256
_NHEADS = 8
_NLAYERS = 6
_B = 100
_NPG = 100
_EPG = 1600
_HD = _NHID // _NHEADS


def _gnn_graph_kernel(src_ref, dst_ref, x_ref, w_ref, as_ref, ad_ref, b_ref,
                      out_ref, acc_ref):

    src = src_ref[0]
    dst = dst_ref[0]

    node_iota = jax.lax.broadcasted_iota(jnp.int32, (_NPG, _EPG), 0)
    st = (node_iota == src).astype(jnp.bfloat16)
    dt = (node_iota == dst).astype(jnp.bfloat16)
    cnt = jax.lax.dot_general(dt, st, (((1,), (1,)), ((), ())),
                              preferred_element_type=jnp.float32)
    eye_r = jax.lax.broadcasted_iota(jnp.int32, (_NPG, _NPG), 0)
    eye_c = jax.lax.broadcasted_iota(jnp.int32, (_NPG, _NPG), 1)
    cnt = cnt + (eye_r == eye_c).astype(jnp.float32)
    mask = cnt > 0.5

    ones_col = jnp.ones((_NPG, 1), dtype=jnp.bfloat16)
    x = x_ref[0]
    for l in range(_NLAYERS):
        if l > 0:
            x = jnp.maximum(x, 0.0)
        w = w_ref[l]
        a_s = as_ref[l]
        a_d = ad_ref[l]
        h = jnp.dot(x.astype(jnp.bfloat16), w,
                    preferred_element_type=jnp.float32)
        hb = h.astype(jnp.bfloat16)
        als_t = jax.lax.dot_general(a_s, hb, (((0,), (1,)), ((), ())),
                                    preferred_element_type=jnp.float32)
        ald = jnp.dot(hb, a_d, preferred_element_type=jnp.float32)
        for hd_i in range(_NHEADS):
            e = ald[:, hd_i:hd_i + 1] + als_t[hd_i:hd_i + 1, :]
            e = jnp.where(e > 0, e, 0.2 * e)
            ee = cnt * jnp.exp(e)
            denom = jnp.sum(ee, axis=1, keepdims=True)
            h_head = hb[:, hd_i * _HD:(hd_i + 1) * _HD]
            agg = jnp.dot(ee.astype(jnp.bfloat16), h_head,
                          preferred_element_type=jnp.float32)
            acc_ref[:, hd_i * _HD:(hd_i + 1) * _HD] = agg / denom
        x = acc_ref[...] + b_ref[l]

    out_ref[0] = jnp.sum(x, axis=0, keepdims=True) * (1.0 / _NPG)


def kernel(nodes, edges, emb,
           W0, as0, ad0, b0, W1, as1, ad1, b1, W2, as2, ad2, b2,
           W3, as3, ad3, b3, W4, as4, ad4, b4, W5, as5, ad5, b5):
    x = emb[nodes.reshape(-1)].reshape(_B, _NPG, _NINP)
    src = edges[:, 0::2].reshape(_B, 1, _EPG)
    dst = edges[:, 1::2].reshape(_B, 1, _EPG)

    w_all = jnp.stack([W0, W1, W2, W3, W4, W5]).astype(jnp.bfloat16)
    eye_h = jnp.eye(_NHEADS, dtype=jnp.float32)
    as_all = jnp.stack([
        jnp.einsum('hj,hk->hjk', a, eye_h).reshape(_NHID, _NHEADS)
        for a in (as0, as1, as2, as3, as4, as5)]).astype(jnp.bfloat16)
    ad_all = jnp.stack([
        jnp.einsum('hj,hk->hjk', a, eye_h).reshape(_NHID, _NHEADS)
        for a in (ad0, ad1, ad2, ad3, ad4, ad5)]).astype(jnp.bfloat16)
    b_all = jnp.stack([b0, b1, b2, b3, b4, b5]).reshape(_NLAYERS, 1, _NHID)

    out = pl.pallas_call(
        _gnn_graph_kernel,
        grid=(_B,),
        in_specs=[
            pl.BlockSpec((1, 1, _EPG), lambda g: (g, 0, 0)),
            pl.BlockSpec((1, 1, _EPG), lambda g: (g, 0, 0)),
            pl.BlockSpec((1, _NPG, _NINP), lambda g: (g, 0, 0)),
            pl.BlockSpec((_NLAYERS, _NINP, _NHID), lambda g: (0, 0, 0)),
            pl.BlockSpec((_NLAYERS, _NHID, _NHEADS), lambda g: (0, 0, 0)),
            pl.BlockSpec((_NLAYERS, _NHID, _NHEADS), lambda g: (0, 0, 0)),
            pl.BlockSpec((_NLAYERS, 1, _NHID), lambda g: (0, 0, 0)),
        ],
        out_specs=pl.BlockSpec((1, 1, _NHID), lambda g: (g, 0, 0)),
        out_shape=jax.ShapeDtypeStruct((_B, 1, _NHID), jnp.float32),
        scratch_shapes=[pltpu.VMEM((_NPG, _NHID), jnp.float32)],
        compiler_params=pltpu.CompilerParams(
            dimension_semantics=('parallel',)),
    )(src, dst, x, w_all, as_all, ad_all, b_all)
    return out.reshape(_B, _NHID)


# speedup vs baseline: 2.5316x; 1.0207x over previous
import jax
import jax.numpy as jnp
from jax.experimental import pallas as pl
from jax.experimental.pallas import tpu as pltpu

_NTOKEN = 50000
_NINP = 256
_NHID = 256
_NHEADS = 8
_NLAYERS = 6
_B = 100
_NPG = 100
_EPG = 1600
_HD = _NHID // _NHEADS


def _gnn_graph_kernel(src_ref, dst_ref, x_ref, w_ref, as_ref, ad_ref, b_ref,
                      out_ref, acc_ref):

    src = src_ref[0]
    dst = dst_ref[0]

    node_iota = jax.lax.broadcasted_iota(jnp.int32, (_NPG, _EPG), 0)
    st = (node_iota == src).astype(jnp.bfloat16)
    dt = (node_iota == dst).astype(jnp.bfloat16)
    cnt = jax.lax.dot_general(dt, st, (((1,), (1,)), ((), ())),
                              preferred_element_type=jnp.float32)
    eye_r = jax.lax.broadcasted_iota(jnp.int32, (_NPG, _NPG), 0)
    eye_c = jax.lax.broadcasted_iota(jnp.int32, (_NPG, _NPG), 1)
    cnt = cnt + (eye_r == eye_c).astype(jnp.float32)
    mask = cnt > 0.5

    ones_col = jnp.ones((_NPG, 1), dtype=jnp.bfloat16)
    x = x_ref[0]
    for l in range(_NLAYERS):
        if l > 0:
            x = jnp.maximum(x, 0.0)
        w = w_ref[l]
        a_s = as_ref[l]
        a_d = ad_ref[l]
        h = jnp.dot(x.astype(jnp.bfloat16), w,
                    preferred_element_type=jnp.float32)
        hb = h.astype(jnp.bfloat16)
        als_t = jax.lax.dot_general(a_s, hb, (((0,), (1,)), ((), ())),
                                    preferred_element_type=jnp.float32)
        ald = jnp.dot(hb, a_d, preferred_element_type=jnp.float32)
        for hd_i in range(_NHEADS):
            e = ald[:, hd_i:hd_i + 1] + als_t[hd_i:hd_i + 1, :]
            e = jnp.where(e > 0, e, 0.2 * e)
            ee = (cnt * jnp.exp(e)).astype(jnp.bfloat16)
            h_aug = jnp.concatenate(
                [hb[:, hd_i * _HD:(hd_i + 1) * _HD], ones_col], axis=1)
            agg = jnp.dot(ee, h_aug, preferred_element_type=jnp.float32)
            acc_ref[:, hd_i * _HD:(hd_i + 1) * _HD] = (
                agg[:, :_HD] / agg[:, _HD:_HD + 1])
        x = acc_ref[...] + b_ref[l]

    out_ref[0] = jnp.sum(x, axis=0, keepdims=True) * (1.0 / _NPG)


def kernel(nodes, edges, emb,
           W0, as0, ad0, b0, W1, as1, ad1, b1, W2, as2, ad2, b2,
           W3, as3, ad3, b3, W4, as4, ad4, b4, W5, as5, ad5, b5):
    x = emb[nodes.reshape(-1)].reshape(_B, _NPG, _NINP)
    src = edges[:, 0::2].reshape(_B, 1, _EPG)
    dst = edges[:, 1::2].reshape(_B, 1, _EPG)

    w_all = jnp.stack([W0, W1, W2, W3, W4, W5]).astype(jnp.bfloat16)
    eye_h = jnp.eye(_NHEADS, dtype=jnp.float32)
    as_all = jnp.stack([
        jnp.einsum('hj,hk->hjk', a, eye_h).reshape(_NHID, _NHEADS)
        for a in (as0, as1, as2, as3, as4, as5)]).astype(jnp.bfloat16)
    ad_all = jnp.stack([
        jnp.einsum('hj,hk->hjk', a, eye_h).reshape(_NHID, _NHEADS)
        for a in (ad0, ad1, ad2, ad3, ad4, ad5)]).astype(jnp.bfloat16)
    b_all = jnp.stack([b0, b1, b2, b3, b4, b5]).reshape(_NLAYERS, 1, _NHID)

    out = pl.pallas_call(
        _gnn_graph_kernel,
        grid=(_B,),
        in_specs=[
            pl.BlockSpec((1, 1, _EPG), lambda g: (g, 0, 0)),
            pl.BlockSpec((1, 1, _EPG), lambda g: (g, 0, 0)),
            pl.BlockSpec((1, _NPG, _NINP), lambda g: (g, 0, 0)),
            pl.BlockSpec((_NLAYERS, _NINP, _NHID), lambda g: (0, 0, 0)),
            pl.BlockSpec((_NLAYERS, _NHID, _NHEADS), lambda g: (0, 0, 0)),
            pl.BlockSpec((_NLAYERS, _NHID, _NHEADS), lambda g: (0, 0, 0)),
            pl.BlockSpec((_NLAYERS, 1, _NHID), lambda g: (0, 0, 0)),
        ],
        out_specs=pl.BlockSpec((1, 1, _NHID), lambda g: (g, 0, 0)),
        out_shape=jax.ShapeDtypeStruct((_B, 1, _NHID), jnp.float32),
        scratch_shapes=[pltpu.VMEM((_NPG, _NHID), jnp.float32)],
        compiler_params=pltpu.CompilerParams(
            dimension_semantics=('parallel',)),
    )(src, dst, x, w_all, as_all, ad_all, b_all)
    return out.reshape(_B, _NHID)
